# baseline (speedup 1.0000x reference)
import dataclasses
import functools

import jax
import jax.numpy as jnp
from jax import lax
from jax.experimental import pallas as pl
from jax.experimental.pallas import tpu as pltpu
from jax.experimental.pallas import tpu_sc as plsc

N = 10000
D = 128
H2 = 64
NC = 2
NS = 16
NW = NC * NS
CHUNK = 128
NP = 10240
ROWS_PER_SUB = NP // NS

_cp_no_layout = pltpu.CompilerParams()
if "needs_layout_passes" in pltpu.CompilerParams.__dataclass_fields__:
    _cp_no_layout = dataclasses.replace(_cp_no_layout,
                                        needs_layout_passes=False)


def _ceil_to(x, m):
    return (x + m - 1) // m * m


def _make_deg_kernel(n_chunks_per_worker):
    C = n_chunks_per_worker
    mesh = plsc.VectorSubcoreMesh(core_axis_name="c", subcore_axis_name="s")

    @functools.partial(
        pl.kernel,
        out_type=jax.ShapeDtypeStruct((NC, NP), jnp.float32),
        mesh=mesh,
        compiler_params=_cp_no_layout,
        scratch_types=[
            pltpu.VMEM((C, CHUNK), jnp.int32),
            pltpu.VMEM((NP,), jnp.float32),
            pltpu.VMEM((NS, ROWS_PER_SUB), jnp.float32),
            pltpu.VMEM_SHARED((NS, NP), jnp.float32),
            pltpu.SemaphoreType.DMA,
        ],
    )
    def deg_kernel(dst_hbm, out_hbm, dstv, hist, red, sbuf, sem):
        cid = lax.axis_index("c")
        sid = lax.axis_index("s")
        wid = sid * NC + cid

        @pl.loop(0, NP // 16)
        def _(i):
            hist[pl.ds(i * 16, 16)] = jnp.zeros((16,), jnp.float32)

        pltpu.sync_copy(dst_hbm.at[pl.ds(wid * C, C)], dstv)
        ones = jnp.ones((16,), jnp.float32)

        @pl.loop(0, C)
        def _(j):
            for k in range(CHUNK // 16):
                ix = dstv[j, pl.ds(k * 16, 16)]
                plsc.addupdate_scatter(hist, [ix], ones)

        pltpu.sync_copy(hist, sbuf.at[sid])
        plsc.subcore_barrier()
        base = sid * ROWS_PER_SUB
        for t in range(NS):
            pltpu.make_async_copy(sbuf.at[t, pl.ds(base, ROWS_PER_SUB)],
                                  red.at[t], sem).start()
        for t in range(NS):
            pltpu.make_async_copy(sbuf.at[t, pl.ds(base, ROWS_PER_SUB)],
                                  red.at[t], sem).wait()

        @pl.loop(0, ROWS_PER_SUB // 16)
        def _(q):
            s = red[0, pl.ds(q * 16, 16)]
            for t in range(1, NS):
                s = s + red[t, pl.ds(q * 16, 16)]
            hist[pl.ds(q * 16, 16)] = s

        pltpu.sync_copy(hist.at[pl.ds(0, ROWS_PER_SUB)],
                        out_hbm.at[cid, pl.ds(base, ROWS_PER_SUB)])

    return deg_kernel


def _make_conv_kernel(n_chunks_per_worker, n_groups=2):
    C = n_chunks_per_worker
    G = C // n_groups
    G2 = 2 * G
    mesh = plsc.VectorSubcoreMesh(core_axis_name="c", subcore_axis_name="s")

    @functools.partial(
        pl.kernel,
        out_type=jax.ShapeDtypeStruct((NC, NP, D), jnp.float32),
        mesh=mesh,
        scratch_types=[
            pltpu.VMEM((G2, CHUNK // 2), jnp.int32),
            pltpu.VMEM((G, CHUNK), jnp.int32),
            pltpu.VMEM((2 * CHUNK, D), jnp.float32),
            pltpu.VMEM((8, D), jnp.float32),
            pltpu.VMEM_SHARED((NP, D), jnp.float32),
            [pltpu.SemaphoreType.DMA] * 5,
        ],
    )
    def conv_kernel(hs_hbm, src_hbm, dst_hbm, out_hbm,
                    srcv, dstv, ring, zv, acc, sems):
        cid = lax.axis_index("c")
        sid = lax.axis_index("s")
        wid = sid * NC + cid
        Q = CHUNK // 2

        @pl.loop(0, 8)
        def _(i):
            for j in range(D // 16):
                zv[i, pl.ds(j * 16, 16)] = jnp.zeros((16,), jnp.float32)

        base = sid * ROWS_PER_SUB

        @pl.loop(0, ROWS_PER_SUB // 8)
        def _(i):
            pltpu.make_async_copy(zv, acc.at[pl.ds(base + i * 8, 8)],
                                  sems[4]).start()

        def g_start(l, q):
            pltpu.make_async_copy(hs_hbm.at[srcv.at[l]],
                                  ring.at[pl.ds(q * Q, Q)], sems[q]).start()

        def g_wait(l, q):
            pltpu.make_async_copy(hs_hbm.at[srcv.at[l]],
                                  ring.at[pl.ds(q * Q, Q)], sems[q]).wait()

        for g in range(n_groups):
            pltpu.sync_copy(src_hbm.at[pl.ds(wid * n_groups * G2 + g * G2,
                                             G2)], srcv)
            pltpu.sync_copy(dst_hbm.at[pl.ds(wid * C + g * G, G)], dstv)
            for q in range(4):
                g_start(q, q)

            if g == 0:
                @pl.loop(0, ROWS_PER_SUB // 8)
                def _(i):
                    pltpu.make_async_copy(zv, acc.at[pl.ds(base + i * 8, 8)],
                                          sems[4]).wait()
                plsc.subcore_barrier()

            @pl.loop(0, G // 2)
            def _(i):
                for h in range(2):
                    q0, q1 = 2 * h, 2 * h + 1
                    g_wait(4 * i + q0, q0)
                    g_wait(4 * i + q1, q1)
                    pltpu.sync_copy(ring.at[pl.ds(h * CHUNK, CHUNK)],
                                    acc.at[dstv.at[2 * i + h]], add=True)

                    @pl.when(i < G // 2 - 1)
                    def _():
                        g_start(4 * i + 4 + q0, q0)
                        g_start(4 * i + 4 + q1, q1)

        plsc.subcore_barrier()
        pltpu.sync_copy(
            acc.at[pl.ds(base, ROWS_PER_SUB)],
            out_hbm.at[cid, pl.ds(base, ROWS_PER_SUB)],
        )

    return conv_kernel


def _mm_scale_kernel(degp_ref, x_ref, w_ref, hs_ref, dinv_ref):
    deg = (degp_ref[0, :N] + degp_ref[1, :N] + 1.0).reshape(N, 1)
    dinv = 1.0 / jnp.sqrt(deg)
    dinv_ref[...] = dinv
    h = jnp.dot(x_ref[...], w_ref[...],
                preferred_element_type=jnp.float32,
                precision=lax.Precision.HIGHEST)
    hs_ref[...] = h * dinv


def _tc_mm_scale(degp, x, w):
    return pl.pallas_call(
        _mm_scale_kernel,
        out_shape=(jax.ShapeDtypeStruct((N, D), jnp.float32),
                   jax.ShapeDtypeStruct((N, 1), jnp.float32)),
    )(degp, x, w)


def _mid_kernel(p_ref, hs_ref, dinv_ref, w_ref, o_ref):
    dinv = dinv_ref[...]
    a1 = dinv * (hs_ref[...] + p_ref[0, :N, :] + p_ref[1, :N, :])
    h = jnp.maximum(a1, 0.0)
    o_ref[...] = dinv * jnp.dot(h, w_ref[...],
                                preferred_element_type=jnp.float32,
                                precision=lax.Precision.HIGHEST)


def _tc_mid(p, hs, dinv, w23):
    return pl.pallas_call(
        _mid_kernel,
        out_shape=jax.ShapeDtypeStruct((N, D), jnp.float32),
    )(p, hs, dinv, w23)


def _final_kernel(p_ref, hs_ref, dinv_ref, eps_ref, o_ref):
    dinv = dinv_ref[...]
    out23 = dinv * (hs_ref[...] + p_ref[0, :N, :] + p_ref[1, :N, :])
    mu = out23[:, :H2]
    logvar = out23[:, H2:]
    o_ref[...] = eps_ref[...] * jnp.exp(logvar) + mu


def _tc_final(p, hs, dinv, eps):
    return pl.pallas_call(
        _final_kernel,
        out_shape=jax.ShapeDtypeStruct((N, H2), jnp.float32),
    )(p, hs, dinv, eps)


def kernel(data, edge_index, W1, W2, W3, eps):
    E = edge_index.shape[1]
    EP = _ceil_to(E, NW * CHUNK * 8)
    n_chunks = EP // (NW * CHUNK)

    src = edge_index[0]
    dst = edge_index[1]
    pad = EP - E
    pad_ar = jnp.arange(pad, dtype=jnp.int32)
    srcp = jnp.concatenate([src, pad_ar % N])
    dstp = jnp.concatenate([dst, N + pad_ar % (NP - N)])
    src2 = srcp.reshape(EP // (CHUNK // 2), CHUNK // 2)
    dst2 = dstp.reshape(EP // CHUNK, CHUNK)

    W23 = jnp.concatenate([W2, W3], axis=1)

    deg_k = _make_deg_kernel(n_chunks)
    conv_k = _make_conv_kernel(n_chunks)

    degp = deg_k(dst2)
    hs1, dinv = _tc_mm_scale(degp, data, W1)

    p1 = conv_k(hs1, src2, dst2)
    hs23 = _tc_mid(p1, hs1, dinv, W23)

    p2 = conv_k(hs23, src2, dst2)
    z = _tc_final(p2, hs23, dinv, eps)
    return z


# speedup vs baseline: 1.0129x; 1.0129x over previous
import dataclasses
import functools

import jax
import jax.numpy as jnp
from jax import lax
from jax.experimental import pallas as pl
from jax.experimental.pallas import tpu as pltpu
from jax.experimental.pallas import tpu_sc as plsc

N = 10000
D = 128
H2 = 64
NC = 2
NS = 16
NW = NC * NS
CHUNK = 128
NP = 10240
ROWS_PER_SUB = NP // NS

_cp_no_layout = pltpu.CompilerParams()
if "needs_layout_passes" in pltpu.CompilerParams.__dataclass_fields__:
    _cp_no_layout = dataclasses.replace(_cp_no_layout,
                                        needs_layout_passes=False)


def _ceil_to(x, m):
    return (x + m - 1) // m * m


def _make_deg_kernel(n_chunks_per_worker):
    C = n_chunks_per_worker
    mesh = plsc.VectorSubcoreMesh(core_axis_name="c", subcore_axis_name="s")

    @functools.partial(
        pl.kernel,
        out_type=jax.ShapeDtypeStruct((NC, NP), jnp.float32),
        mesh=mesh,
        compiler_params=_cp_no_layout,
        scratch_types=[
            pltpu.VMEM((C, CHUNK), jnp.int32),
            pltpu.VMEM((NP,), jnp.float32),
            pltpu.VMEM((NS, ROWS_PER_SUB), jnp.float32),
            pltpu.VMEM_SHARED((NS, NP), jnp.float32),
            pltpu.SemaphoreType.DMA,
        ],
    )
    def deg_kernel(dst_hbm, out_hbm, dstv, hist, red, sbuf, sem):
        cid = lax.axis_index("c")
        sid = lax.axis_index("s")
        wid = sid * NC + cid

        @pl.loop(0, NP // 16)
        def _(i):
            hist[pl.ds(i * 16, 16)] = jnp.zeros((16,), jnp.float32)

        pltpu.sync_copy(dst_hbm.at[pl.ds(wid * C, C)], dstv)
        ones = jnp.ones((16,), jnp.float32)

        @pl.loop(0, C)
        def _(j):
            for k in range(CHUNK // 16):
                ix = dstv[j, pl.ds(k * 16, 16)]
                plsc.addupdate_scatter(hist, [ix], ones)

        pltpu.sync_copy(hist, sbuf.at[sid])
        plsc.subcore_barrier()
        base = sid * ROWS_PER_SUB
        for t in range(NS):
            pltpu.make_async_copy(sbuf.at[t, pl.ds(base, ROWS_PER_SUB)],
                                  red.at[t], sem).start()
        for t in range(NS):
            pltpu.make_async_copy(sbuf.at[t, pl.ds(base, ROWS_PER_SUB)],
                                  red.at[t], sem).wait()

        @pl.loop(0, ROWS_PER_SUB // 16)
        def _(q):
            s = red[0, pl.ds(q * 16, 16)]
            for t in range(1, NS):
                s = s + red[t, pl.ds(q * 16, 16)]
            hist[pl.ds(q * 16, 16)] = s

        pltpu.sync_copy(hist.at[pl.ds(0, ROWS_PER_SUB)],
                        out_hbm.at[cid, pl.ds(base, ROWS_PER_SUB)])

    return deg_kernel


def _make_conv_kernel(n_chunks_per_worker, n_groups=2):
    C = n_chunks_per_worker
    G = C // n_groups
    G2 = 2 * G
    mesh = plsc.VectorSubcoreMesh(core_axis_name="c", subcore_axis_name="s")

    @functools.partial(
        pl.kernel,
        out_type=jax.ShapeDtypeStruct((NC, NP, D), jnp.float32),
        mesh=mesh,
        scratch_types=[
            pltpu.VMEM((G2, CHUNK // 2), jnp.int32),
            pltpu.VMEM((G, CHUNK), jnp.int32),
            pltpu.VMEM((2 * CHUNK, D), jnp.float32),
            pltpu.VMEM((8, D), jnp.float32),
            pltpu.VMEM_SHARED((NP, D), jnp.float32),
            [pltpu.SemaphoreType.DMA] * 5,
        ],
    )
    def conv_kernel(hs_hbm, src_hbm, dst_hbm, out_hbm,
                    srcv, dstv, ring, zv, acc, sems):
        cid = lax.axis_index("c")
        sid = lax.axis_index("s")
        wid = sid * NC + cid
        Q = CHUNK // 2

        @pl.loop(0, 8)
        def _(i):
            for j in range(D // 16):
                zv[i, pl.ds(j * 16, 16)] = jnp.zeros((16,), jnp.float32)

        base = sid * ROWS_PER_SUB

        @pl.loop(0, ROWS_PER_SUB // 8)
        def _(i):
            pltpu.make_async_copy(zv, acc.at[pl.ds(base + i * 8, 8)],
                                  sems[4]).start()

        def g_start(l, q):
            pltpu.make_async_copy(hs_hbm.at[srcv.at[l]],
                                  ring.at[pl.ds(q * Q, Q)], sems[q]).start()

        def g_wait(l, q):
            pltpu.make_async_copy(hs_hbm.at[srcv.at[l]],
                                  ring.at[pl.ds(q * Q, Q)], sems[q]).wait()

        for g in range(n_groups):
            pltpu.sync_copy(src_hbm.at[pl.ds(wid * n_groups * G2 + g * G2,
                                             G2)], srcv)
            pltpu.sync_copy(dst_hbm.at[pl.ds(wid * C + g * G, G)], dstv)
            for q in range(4):
                g_start(q, q)

            if g == 0:
                @pl.loop(0, ROWS_PER_SUB // 8)
                def _(i):
                    pltpu.make_async_copy(zv, acc.at[pl.ds(base + i * 8, 8)],
                                          sems[4]).wait()
                plsc.subcore_barrier()

            @pl.loop(0, G // 2)
            def _(i):
                for h in range(2):
                    q0, q1 = 2 * h, 2 * h + 1
                    g_wait(4 * i + q0, q0)
                    g_wait(4 * i + q1, q1)
                    pltpu.sync_copy(ring.at[pl.ds(h * CHUNK, CHUNK)],
                                    acc.at[dstv.at[2 * i + h]], add=True)

                    @pl.when(i < G // 2 - 1)
                    def _():
                        g_start(4 * i + 4 + q0, q0)
                        g_start(4 * i + 4 + q1, q1)

        plsc.subcore_barrier()
        pltpu.sync_copy(
            acc.at[pl.ds(base, ROWS_PER_SUB)],
            out_hbm.at[cid, pl.ds(base, ROWS_PER_SUB)],
        )

    return conv_kernel


def _mm_kernel(x_ref, w_ref, o_ref):
    o_ref[...] = jnp.dot(x_ref[...], w_ref[...],
                         preferred_element_type=jnp.float32,
                         precision=lax.Precision.HIGHEST)


def _tc_matmul(x, w):
    return pl.pallas_call(
        _mm_kernel,
        out_shape=jax.ShapeDtypeStruct((x.shape[0], w.shape[1]), jnp.float32),
    )(x, w)


def _scale_kernel(degp_ref, h_ref, hs_ref, dinv_ref):
    deg = (degp_ref[0, :N] + degp_ref[1, :N] + 1.0).reshape(N, 1)
    dinv = 1.0 / jnp.sqrt(deg)
    dinv_ref[...] = dinv
    hs_ref[...] = h_ref[...] * dinv


def _tc_scale(degp, h):
    return pl.pallas_call(
        _scale_kernel,
        out_shape=(jax.ShapeDtypeStruct((N, D), jnp.float32),
                   jax.ShapeDtypeStruct((N, 1), jnp.float32)),
    )(degp, h)


def _mid_kernel(p_ref, hs_ref, dinv_ref, w_ref, o_ref):
    dinv = dinv_ref[...]
    a1 = dinv * (hs_ref[...] + p_ref[0, :N, :] + p_ref[1, :N, :])
    h = jnp.maximum(a1, 0.0)
    o_ref[...] = dinv * jnp.dot(h, w_ref[...],
                                preferred_element_type=jnp.float32,
                                precision=lax.Precision.HIGHEST)


def _tc_mid(p, hs, dinv, w23):
    return pl.pallas_call(
        _mid_kernel,
        out_shape=jax.ShapeDtypeStruct((N, D), jnp.float32),
    )(p, hs, dinv, w23)


def _final_kernel(p_ref, hs_ref, dinv_ref, eps_ref, o_ref):
    dinv = dinv_ref[...]
    out23 = dinv * (hs_ref[...] + p_ref[0, :N, :] + p_ref[1, :N, :])
    mu = out23[:, :H2]
    logvar = out23[:, H2:]
    o_ref[...] = eps_ref[...] * jnp.exp(logvar) + mu


def _tc_final(p, hs, dinv, eps):
    return pl.pallas_call(
        _final_kernel,
        out_shape=jax.ShapeDtypeStruct((N, H2), jnp.float32),
    )(p, hs, dinv, eps)


def kernel(data, edge_index, W1, W2, W3, eps):
    E = edge_index.shape[1]
    EP = _ceil_to(E, NW * CHUNK * 8)
    n_chunks = EP // (NW * CHUNK)

    src = edge_index[0]
    dst = edge_index[1]
    pad = EP - E
    pad_ar = jnp.arange(pad, dtype=jnp.int32)
    srcp = jnp.concatenate([src, pad_ar % N])
    dstp = jnp.concatenate([dst, N + pad_ar % (NP - N)])
    src2 = srcp.reshape(EP // (CHUNK // 2), CHUNK // 2)
    dst2 = dstp.reshape(EP // CHUNK, CHUNK)

    W23 = jnp.concatenate([W2, W3], axis=1)

    deg_k = _make_deg_kernel(n_chunks)
    conv_k = _make_conv_kernel(n_chunks)

    degp = deg_k(dst2)
    h1 = _tc_matmul(data, W1)
    hs1, dinv = _tc_scale(degp, h1)

    p1 = conv_k(hs1, src2, dst2)
    hs23 = _tc_mid(p1, hs1, dinv, W23)

    p2 = conv_k(hs23, src2, dst2)
    z = _tc_final(p2, hs23, dinv, eps)
    return z
